# baseline (speedup 1.0000x reference)
import functools

import jax
import jax.numpy as jnp
from jax import lax
from jax.experimental import pallas as pl
from jax.experimental.pallas import tpu as pltpu
from jax.experimental.pallas import tpu_sc as plsc

_N_FIELDS = 26
_VOCAB = 100000
_EMB = 32
_N_NUM = 13
_B = 16384
_NC, _NS = 2, 16
_NW = _NC * _NS
_BPW = _B // _NW
_IDXW = 128
_Q = _VOCAB // 4
_G = _N_FIELDS * _Q
_CH = 256
_NH = _BPW // _CH
_IPW = _N_FIELDS * (_BPW // _IDXW)


def _repack_body(t_ref, eye_ref, out_ref):
    eye = eye_ref[...]
    for s in range(4):
        out_ref[:, s * _EMB:(s + 1) * _EMB] = lax.dot_general(
            t_ref[0, :, s * _Q:(s + 1) * _Q], eye,
            dimension_numbers=(((0,), (0,)), ((), ())),
            preferred_element_type=jnp.float32)


def _repack(tabT):
    return pl.pallas_call(
        _repack_body,
        grid=(_N_FIELDS,),
        in_specs=[pl.BlockSpec((1, _EMB, _VOCAB), lambda i: (i, 0, 0)),
                  pl.BlockSpec((_EMB, _EMB), lambda i: (0, 0))],
        out_specs=pl.BlockSpec((_Q, 128), lambda i: (i, 0)),
        out_shape=jax.ShapeDtypeStruct((_G, 128), jnp.float32),
        compiler_params=pltpu.CompilerParams(
            dimension_semantics=("arbitrary",),
            vmem_limit_bytes=110 * 1024 * 1024),
    )(tabT, jnp.eye(_EMB, dtype=jnp.float32))


def _gather_body(tab_ref, gidx_ref, sidx_ref, out_ref,
                 gidx_v, sidx_v, rows_v, ob_v, sem0, sem1):
    wid = lax.axis_index("c") * _NS + lax.axis_index("s")
    pltpu.sync_copy(gidx_ref.at[wid], gidx_v)
    pltpu.sync_copy(sidx_ref.at[wid], sidx_v)
    sems = (sem0, sem1)

    def fire(k, h, buf):
        for j in range(_CH // _IDXW):
            pltpu.async_copy(
                tab_ref.at[gidx_v.at[k * 4 + h * 2 + j]],
                rows_v.at[buf, pl.ds(j * _IDXW, _IDXW)],
                sems[buf])

    def drain(buf):
        for j in range(_CH // _IDXW):
            pltpu.make_async_copy(
                tab_ref.at[pl.ds(0, _IDXW)],
                rows_v.at[buf, pl.ds(j * _IDXW, _IDXW)],
                sems[buf]).wait()

    def select(k, h, buf):
        rows_p = rows_v.at[buf]
        ob_p = ob_v.at[buf]

        def body(g):
            sv = sidx_v[k * 4 + h * 2 + g // 8, pl.ds((g % 8) * 16, 16)]
            for q in range(16):
                s = sv[q]
                off = s * _EMB
                i_loc = g * 16 + q
                row = 4 * g + q // 4
                for half in range(2):
                    ob_p[row, pl.ds((q % 4) * _EMB + 16 * half, 16)] = (
                        rows_p[i_loc, pl.ds(off + 16 * half, 16)])

        pl.loop(0, _CH // 16)(body)

    def copy_out(k, h, buf):
        pltpu.sync_copy(
            ob_v.at[buf],
            out_ref.at[k].at[pl.ds(wid * (_BPW // 4) + h * (_CH // 4),
                                   _CH // 4)])

    fire(0, 0, 0)

    def kbody(k):
        fire(k, 1, 1)
        drain(0)
        select(k, 0, 0)
        copy_out(k, 0, 0)

        @pl.when(k < _N_FIELDS - 1)
        def _():
            fire(k + 1, 0, 0)

        drain(1)
        select(k, 1, 1)
        copy_out(k, 1, 1)

    pl.loop(0, _N_FIELDS)(kbody)


def _sc_gather(tab128, gidxr, sidxr):
    mesh = plsc.VectorSubcoreMesh(core_axis_name="c", subcore_axis_name="s")
    k = pl.kernel(
        _gather_body,
        out_type=jax.ShapeDtypeStruct((_N_FIELDS, _B // 4, 128), jnp.float32),
        mesh=mesh,
        scratch_types=[
            pltpu.VMEM((_IPW, _IDXW), jnp.int32),
            pltpu.VMEM((_IPW, _IDXW), jnp.int32),
            pltpu.VMEM((2, _CH, 128), jnp.float32),
            pltpu.VMEM((2, _CH // 4, 128), jnp.float32),
            pltpu.SemaphoreType.DMA,
            pltpu.SemaphoreType.DMA,
        ],
        compiler_params=pltpu.CompilerParams(use_tc_tiling_on_sc=True),
    )
    return k(tab128, gidxr, sidxr)


_BLKP = 256


def _mlp_body(xw_ref, emb_ref, bd1_ref, bd2_ref, b2_ref, bd3_ref, b3_ref,
              out_ref):
    h = xw_ref[...]
    for k in range(_N_FIELDS):
        h = h + jnp.dot(emb_ref[k], bd1_ref[k],
                        preferred_element_type=jnp.float32)
    h = jnp.maximum(h, 0.0)
    h = jnp.maximum(
        jnp.dot(h, bd2_ref[...], preferred_element_type=jnp.float32)
        + b2_ref[...], 0.0)
    out_ref[...] = (jnp.dot(h, bd3_ref[...], preferred_element_type=jnp.float32)
                    + b3_ref[...])


def _mlp(xw, emb4, bd1, bd2, b2p, bd3, b3p):
    grid = (_B // 4 // _BLKP,)
    return pl.pallas_call(
        _mlp_body,
        grid=grid,
        in_specs=[
            pl.BlockSpec((_BLKP, 512), lambda i: (i, 0)),
            pl.BlockSpec((_N_FIELDS, _BLKP, 128), lambda i: (0, i, 0)),
            pl.BlockSpec((_N_FIELDS, 128, 512), lambda i: (0, 0, 0)),
            pl.BlockSpec((512, 256), lambda i: (0, 0)),
            pl.BlockSpec((1, 256), lambda i: (0, 0)),
            pl.BlockSpec((256, 4), lambda i: (0, 0)),
            pl.BlockSpec((1, 4), lambda i: (0, 0)),
        ],
        out_specs=pl.BlockSpec((_BLKP, 4), lambda i: (i, 0)),
        out_shape=jax.ShapeDtypeStruct((_B // 4, 4), jnp.float32),
        compiler_params=pltpu.CompilerParams(
            dimension_semantics=("parallel",)),
    )(xw, emb4, bd1, bd2, b2p, bd3, b3p)


def kernel(x_num, x_cat, tables, W1, b1, W2, b2, W3, b3):
    tabT = tables.transpose(0, 2, 1)
    tab128 = _repack(tabT)

    v = x_cat.astype(jnp.int32)
    offs = (jnp.arange(_N_FIELDS, dtype=jnp.int32) * _Q)[None, :]
    gidx = offs + v % _Q
    sidx = v // _Q

    def to_worker_major(a):
        return (a.reshape(_NW, _BPW // _IDXW, _IDXW, _N_FIELDS)
                .transpose(0, 3, 1, 2)
                .reshape(_NW, _IPW, _IDXW))

    emb4 = _sc_gather(tab128, to_worker_major(gidx), to_worker_major(sidx))

    eye4 = jnp.eye(4, dtype=jnp.float32)
    W1e = W1[_N_NUM:].reshape(_N_FIELDS, _EMB, 128)
    bd1 = (eye4[None, :, None, :, None]
           * W1e[:, None, :, None, :]).reshape(_N_FIELDS, 128, 512)
    bd2 = (eye4[:, None, :, None]
           * W2[None, :, None, :]).reshape(512, 256)
    bd3 = (eye4[:, None, :, None]
           * W3[None, :, None, :]).reshape(256, 4)
    xw = (x_num @ W1[:_N_NUM] + b1).reshape(_B // 4, 512)
    b2p = jnp.tile(b2, 4).reshape(1, 256)
    b3p = jnp.tile(b3, 4).reshape(1, 4)

    out4 = _mlp(xw, emb4, bd1, bd2, b2p, bd3, b3p)
    return out4.reshape(_B, 1)


# speedup vs baseline: 2.0216x; 2.0216x over previous
import functools

import jax
import jax.numpy as jnp
from jax import lax
from jax.experimental import pallas as pl
from jax.experimental.pallas import tpu as pltpu
from jax.experimental.pallas import tpu_sc as plsc

_N_FIELDS = 26
_VOCAB = 100000
_EMB = 32
_N_NUM = 13
_B = 16384
_NC, _NS = 2, 16
_NW = _NC * _NS
_BPW = _B // _NW
_IDXW = 128
_Q = _VOCAB // 4
_G = _N_FIELDS * _Q
_CH = 256
_NH = _BPW // _CH
_IPW = _N_FIELDS * (_BPW // _IDXW)


_MCH = 5000


def _repack_body(t_ref, out_ref):
    for c in range(_Q // _MCH):
        lo = c * _MCH
        u = jnp.concatenate(
            [t_ref[0, :, s * _Q + lo:s * _Q + lo + _MCH] for s in range(4)],
            axis=0)
        out_ref[pl.ds(lo, _MCH), :] = jnp.transpose(u, (1, 0))


def _repack(tabT):
    return pl.pallas_call(
        _repack_body,
        grid=(_N_FIELDS,),
        in_specs=[pl.BlockSpec((1, _EMB, _VOCAB), lambda i: (i, 0, 0))],
        out_specs=pl.BlockSpec((_Q, 128), lambda i: (i, 0)),
        out_shape=jax.ShapeDtypeStruct((_G, 128), jnp.float32),
        compiler_params=pltpu.CompilerParams(
            dimension_semantics=("arbitrary",),
            vmem_limit_bytes=110 * 1024 * 1024),
    )(tabT)


def _gather_body(tab_ref, gidx_ref, sidx_ref, out_ref,
                 gidx_v, sidx_v, rows_v, ob_v, sem0, sem1):
    wid = lax.axis_index("c") * _NS + lax.axis_index("s")
    pltpu.sync_copy(gidx_ref.at[wid], gidx_v)
    pltpu.sync_copy(sidx_ref.at[wid], sidx_v)
    sems = (sem0, sem1)

    def fire(k, h, buf):
        for j in range(_CH // _IDXW):
            pltpu.async_copy(
                tab_ref.at[gidx_v.at[k * 4 + h * 2 + j]],
                rows_v.at[buf, pl.ds(j * _IDXW, _IDXW)],
                sems[buf])

    def drain(buf):
        for j in range(_CH // _IDXW):
            pltpu.make_async_copy(
                tab_ref.at[pl.ds(0, _IDXW)],
                rows_v.at[buf, pl.ds(j * _IDXW, _IDXW)],
                sems[buf]).wait()

    def select(k, h, buf):
        rows_p = rows_v.at[buf]
        ob_p = ob_v.at[buf]

        def body(g):
            sv = sidx_v[k * 4 + h * 2 + g // 8, pl.ds((g % 8) * 16, 16)]
            for q in range(16):
                s = sv[q]
                off = s * _EMB
                i_loc = g * 16 + q
                row = 4 * g + q // 4
                for half in range(2):
                    ob_p[row, pl.ds((q % 4) * _EMB + 16 * half, 16)] = (
                        rows_p[i_loc, pl.ds(off + 16 * half, 16)])

        pl.loop(0, _CH // 16)(body)

    def copy_out(k, h, buf):
        pltpu.sync_copy(
            ob_v.at[buf],
            out_ref.at[k].at[pl.ds(wid * (_BPW // 4) + h * (_CH // 4),
                                   _CH // 4)])

    fire(0, 0, 0)

    def kbody(k):
        fire(k, 1, 1)
        drain(0)
        select(k, 0, 0)
        copy_out(k, 0, 0)

        @pl.when(k < _N_FIELDS - 1)
        def _():
            fire(k + 1, 0, 0)

        drain(1)
        select(k, 1, 1)
        copy_out(k, 1, 1)

    pl.loop(0, _N_FIELDS)(kbody)


def _sc_gather(tab128, gidxr, sidxr):
    mesh = plsc.VectorSubcoreMesh(core_axis_name="c", subcore_axis_name="s")
    k = pl.kernel(
        _gather_body,
        out_type=jax.ShapeDtypeStruct((_N_FIELDS, _B // 4, 128), jnp.float32),
        mesh=mesh,
        scratch_types=[
            pltpu.VMEM((_IPW, _IDXW), jnp.int32),
            pltpu.VMEM((_IPW, _IDXW), jnp.int32),
            pltpu.VMEM((2, _CH, 128), jnp.float32),
            pltpu.VMEM((2, _CH // 4, 128), jnp.float32),
            pltpu.SemaphoreType.DMA,
            pltpu.SemaphoreType.DMA,
        ],
        compiler_params=pltpu.CompilerParams(use_tc_tiling_on_sc=True),
    )
    return k(tab128, gidxr, sidxr)


_BLKP = 256


def _mlp_body(xw_ref, emb_ref, bd1_ref, bd2_ref, b2_ref, bd3_ref, b3_ref,
              out_ref):
    h = xw_ref[...]
    for k in range(_N_FIELDS):
        h = h + jnp.dot(emb_ref[k], bd1_ref[k],
                        preferred_element_type=jnp.float32)
    h = jnp.maximum(h, 0.0)
    h = jnp.maximum(
        jnp.dot(h, bd2_ref[...], preferred_element_type=jnp.float32)
        + b2_ref[...], 0.0)
    out_ref[...] = (jnp.dot(h, bd3_ref[...], preferred_element_type=jnp.float32)
                    + b3_ref[...])


def _mlp(xw, emb4, bd1, bd2, b2p, bd3, b3p):
    grid = (_B // 4 // _BLKP,)
    return pl.pallas_call(
        _mlp_body,
        grid=grid,
        in_specs=[
            pl.BlockSpec((_BLKP, 512), lambda i: (i, 0)),
            pl.BlockSpec((_N_FIELDS, _BLKP, 128), lambda i: (0, i, 0)),
            pl.BlockSpec((_N_FIELDS, 128, 512), lambda i: (0, 0, 0)),
            pl.BlockSpec((512, 256), lambda i: (0, 0)),
            pl.BlockSpec((1, 256), lambda i: (0, 0)),
            pl.BlockSpec((256, 4), lambda i: (0, 0)),
            pl.BlockSpec((1, 4), lambda i: (0, 0)),
        ],
        out_specs=pl.BlockSpec((_BLKP, 4), lambda i: (i, 0)),
        out_shape=jax.ShapeDtypeStruct((_B // 4, 4), jnp.float32),
        compiler_params=pltpu.CompilerParams(
            dimension_semantics=("parallel",)),
    )(xw, emb4, bd1, bd2, b2p, bd3, b3p)


def kernel(x_num, x_cat, tables, W1, b1, W2, b2, W3, b3):
    tabT = tables.transpose(0, 2, 1)
    tab128 = _repack(tabT)

    v = x_cat.astype(jnp.int32)
    offs = (jnp.arange(_N_FIELDS, dtype=jnp.int32) * _Q)[None, :]
    gidx = offs + v % _Q
    sidx = v // _Q

    def to_worker_major(a):
        return (a.reshape(_NW, _BPW // _IDXW, _IDXW, _N_FIELDS)
                .transpose(0, 3, 1, 2)
                .reshape(_NW, _IPW, _IDXW))

    emb4 = _sc_gather(tab128, to_worker_major(gidx), to_worker_major(sidx))

    eye4 = jnp.eye(4, dtype=jnp.float32)
    W1e = W1[_N_NUM:].reshape(_N_FIELDS, _EMB, 128)
    bd1 = (eye4[None, :, None, :, None]
           * W1e[:, None, :, None, :]).reshape(_N_FIELDS, 128, 512)
    bd2 = (eye4[:, None, :, None]
           * W2[None, :, None, :]).reshape(512, 256)
    bd3 = (eye4[:, None, :, None]
           * W3[None, :, None, :]).reshape(256, 4)
    xw = (x_num @ W1[:_N_NUM] + b1).reshape(_B // 4, 512)
    b2p = jnp.tile(b2, 4).reshape(1, 256)
    b3p = jnp.tile(b3, 4).reshape(1, 4)

    out4 = _mlp(xw, emb4, bd1, bd2, b2p, bd3, b3p)
    return out4.reshape(_B, 1)


# speedup vs baseline: 2.0688x; 1.0233x over previous
import functools

import jax
import jax.numpy as jnp
from jax import lax
from jax.experimental import pallas as pl
from jax.experimental.pallas import tpu as pltpu
from jax.experimental.pallas import tpu_sc as plsc

_N_FIELDS = 26
_VOCAB = 100000
_EMB = 32
_N_NUM = 13
_B = 16384
_NC, _NS = 2, 16
_NW = _NC * _NS
_BPW = _B // _NW
_IDXW = 128
_Q = _VOCAB // 4
_G = _N_FIELDS * _Q
_CH = 256
_NH = _BPW // _CH
_IPW = _N_FIELDS * (_BPW // _IDXW)


_MCH = 5000


def _repack_body(t_ref, out_ref):
    for c in range(_Q // _MCH):
        lo = c * _MCH
        u = jnp.concatenate(
            [t_ref[0, :, s * _Q + lo:s * _Q + lo + _MCH] for s in range(4)],
            axis=0)
        out_ref[pl.ds(lo, _MCH), :] = jnp.transpose(u, (1, 0))


def _repack(tabT, base, nf):
    return pl.pallas_call(
        _repack_body,
        grid=(nf,),
        in_specs=[pl.BlockSpec((1, _EMB, _VOCAB), lambda i: (base + i, 0, 0))],
        out_specs=pl.BlockSpec((_Q, 128), lambda i: (i, 0)),
        out_shape=jax.ShapeDtypeStruct((nf * _Q, 128), jnp.float32),
        compiler_params=pltpu.CompilerParams(
            dimension_semantics=("arbitrary",),
            vmem_limit_bytes=110 * 1024 * 1024),
    )(tabT)


def _gather_body(nf, tab_ref, gidx_ref, sidx_ref, out_ref,
                 gidx_v, sidx_v, rows_v, ob_v, sem0, sem1):
    wid = lax.axis_index("c") * _NS + lax.axis_index("s")
    pltpu.sync_copy(gidx_ref.at[wid], gidx_v)
    pltpu.sync_copy(sidx_ref.at[wid], sidx_v)
    sems = (sem0, sem1)

    def fire(k, h, buf):
        for j in range(_CH // _IDXW):
            pltpu.async_copy(
                tab_ref.at[gidx_v.at[k * 4 + h * 2 + j]],
                rows_v.at[buf, pl.ds(j * _IDXW, _IDXW)],
                sems[buf])

    def drain(buf):
        for j in range(_CH // _IDXW):
            pltpu.make_async_copy(
                tab_ref.at[pl.ds(0, _IDXW)],
                rows_v.at[buf, pl.ds(j * _IDXW, _IDXW)],
                sems[buf]).wait()

    def select(k, h, buf):
        rows_p = rows_v.at[buf]
        ob_p = ob_v.at[buf]

        def body(g):
            sv = sidx_v[k * 4 + h * 2 + g // 8, pl.ds((g % 8) * 16, 16)]
            for q in range(16):
                s = sv[q]
                off = s * _EMB
                i_loc = g * 16 + q
                row = 4 * g + q // 4
                for half in range(2):
                    ob_p[row, pl.ds((q % 4) * _EMB + 16 * half, 16)] = (
                        rows_p[i_loc, pl.ds(off + 16 * half, 16)])

        pl.loop(0, _CH // 16)(body)

    def copy_out(k, h, buf):
        pltpu.sync_copy(
            ob_v.at[buf],
            out_ref.at[k].at[pl.ds(wid * (_BPW // 4) + h * (_CH // 4),
                                   _CH // 4)])

    fire(0, 0, 0)

    def kbody(k):
        fire(k, 1, 1)
        drain(0)
        select(k, 0, 0)
        copy_out(k, 0, 0)

        @pl.when(k < nf - 1)
        def _():
            fire(k + 1, 0, 0)

        drain(1)
        select(k, 1, 1)
        copy_out(k, 1, 1)

    pl.loop(0, nf)(kbody)


def _sc_gather(tab128, gidxr, sidxr, nf):
    mesh = plsc.VectorSubcoreMesh(core_axis_name="c", subcore_axis_name="s")
    ipw = nf * (_BPW // _IDXW)
    k = pl.kernel(
        functools.partial(_gather_body, nf),
        out_type=jax.ShapeDtypeStruct((nf, _B // 4, 128), jnp.float32),
        mesh=mesh,
        scratch_types=[
            pltpu.VMEM((ipw, _IDXW), jnp.int32),
            pltpu.VMEM((ipw, _IDXW), jnp.int32),
            pltpu.VMEM((2, _CH, 128), jnp.float32),
            pltpu.VMEM((2, _CH // 4, 128), jnp.float32),
            pltpu.SemaphoreType.DMA,
            pltpu.SemaphoreType.DMA,
        ],
        compiler_params=pltpu.CompilerParams(use_tc_tiling_on_sc=True),
    )
    return k(tab128, gidxr, sidxr)


_BLKP = 256


def _mlp_body(xw_ref, emba_ref, embb_ref, bd1_ref, bd2_ref, b2_ref, bd3_ref,
              b3_ref, out_ref):
    h = xw_ref[...]
    nh = _N_FIELDS // 2
    for k in range(nh):
        h = h + jnp.dot(emba_ref[k], bd1_ref[k],
                        preferred_element_type=jnp.float32)
        h = h + jnp.dot(embb_ref[k], bd1_ref[nh + k],
                        preferred_element_type=jnp.float32)
    h = jnp.maximum(h, 0.0)
    h = jnp.maximum(
        jnp.dot(h, bd2_ref[...], preferred_element_type=jnp.float32)
        + b2_ref[...], 0.0)
    out_ref[...] = (jnp.dot(h, bd3_ref[...], preferred_element_type=jnp.float32)
                    + b3_ref[...])


def _mlp(xw, emb4a, emb4b, bd1, bd2, b2p, bd3, b3p):
    grid = (_B // 4 // _BLKP,)
    nh = _N_FIELDS // 2
    return pl.pallas_call(
        _mlp_body,
        grid=grid,
        in_specs=[
            pl.BlockSpec((_BLKP, 512), lambda i: (i, 0)),
            pl.BlockSpec((nh, _BLKP, 128), lambda i: (0, i, 0)),
            pl.BlockSpec((nh, _BLKP, 128), lambda i: (0, i, 0)),
            pl.BlockSpec((_N_FIELDS, 128, 512), lambda i: (0, 0, 0)),
            pl.BlockSpec((512, 256), lambda i: (0, 0)),
            pl.BlockSpec((1, 256), lambda i: (0, 0)),
            pl.BlockSpec((256, 4), lambda i: (0, 0)),
            pl.BlockSpec((1, 4), lambda i: (0, 0)),
        ],
        out_specs=pl.BlockSpec((_BLKP, 4), lambda i: (i, 0)),
        out_shape=jax.ShapeDtypeStruct((_B // 4, 4), jnp.float32),
        compiler_params=pltpu.CompilerParams(
            dimension_semantics=("parallel",)),
    )(xw, emb4a, emb4b, bd1, bd2, b2p, bd3, b3p)


def kernel(x_num, x_cat, tables, W1, b1, W2, b2, W3, b3):
    tabT = tables.transpose(0, 2, 1)
    nh = _N_FIELDS // 2
    v = x_cat.astype(jnp.int32)
    offs = (jnp.arange(nh, dtype=jnp.int32) * _Q)[None, :]

    def to_worker_major(a):
        return (a.reshape(_NW, _BPW // _IDXW, _IDXW, nh)
                .transpose(0, 3, 1, 2)
                .reshape(_NW, nh * (_BPW // _IDXW), _IDXW))

    embs = []
    for hh in range(2):
        vh = v[:, hh * nh:(hh + 1) * nh]
        gidx = offs + vh % _Q
        sidx = vh // _Q
        tab_h = _repack(tabT, hh * nh, nh)
        embs.append(_sc_gather(tab_h, to_worker_major(gidx),
                               to_worker_major(sidx), nh))

    eye4 = jnp.eye(4, dtype=jnp.float32)
    W1e = W1[_N_NUM:].reshape(_N_FIELDS, _EMB, 128)
    bd1 = (eye4[None, :, None, :, None]
           * W1e[:, None, :, None, :]).reshape(_N_FIELDS, 128, 512)
    bd2 = (eye4[:, None, :, None]
           * W2[None, :, None, :]).reshape(512, 256)
    bd3 = (eye4[:, None, :, None]
           * W3[None, :, None, :]).reshape(256, 4)
    xw = (x_num @ W1[:_N_NUM] + b1).reshape(_B // 4, 512)
    b2p = jnp.tile(b2, 4).reshape(1, 256)
    b3p = jnp.tile(b3, 4).reshape(1, 4)

    out4 = _mlp(xw, embs[0], embs[1], bd1, bd2, b2p, bd3, b3p)
    return out4.reshape(_B, 1)
